# Initial kernel scaffold; baseline (speedup 1.0000x reference)
import jax
import jax.numpy as jnp
from jax.experimental import pallas as pl


def kernel(x, edge_index, W1, b1, W2, b2, Wl, bl):
    raise NotImplementedError("write your pallas kernel here")


# speedup vs baseline: 19.3242x; 19.3242x over previous
import functools

import jax
import jax.numpy as jnp
from jax import lax
from jax.experimental import pallas as pl
from jax.experimental.pallas import tpu as pltpu
from jax.experimental.pallas import tpu_sc as plsc

N_NODES = 10000
N_EDGES = 320000
D = 128

NC = 2
NS = 16
NW = NC * NS
E_PER_TILE = N_EDGES // NW
GB = 80
STEPS = E_PER_TILE // GB
N_ACC = 10240
ROWS_PER_TILE = N_ACC // NS
DEG_STEPS = E_PER_TILE // 16

_mesh = lambda: plsc.VectorSubcoreMesh(core_axis_name="c", subcore_axis_name="s")


@functools.partial(
    pl.kernel,
    mesh=_mesh(),
    out_type=jax.ShapeDtypeStruct((NW, 1, N_NODES), jnp.float32),
    scratch_types=[
        pltpu.VMEM((DEG_STEPS, 16), jnp.int32),
        pltpu.VMEM((1, N_NODES), jnp.float32),
    ],
    compiler_params=pltpu.CompilerParams(needs_layout_passes=False),
)
def _deg_sc(dst_hbm, out_hbm, idx_v, deg_v):
    c = lax.axis_index("c")
    s = lax.axis_index("s")
    wid = c * NS + s
    pltpu.sync_copy(dst_hbm.at[wid], idx_v)

    zeros16 = jnp.zeros((16,), jnp.float32)

    def zbody(i, _):
        deg_v[0, pl.ds(i * 16, 16)] = zeros16
        return 0

    lax.fori_loop(0, N_NODES // 16, zbody, 0)

    ones16 = jnp.ones((16,), jnp.float32)
    zeros16i = jnp.zeros((16,), jnp.int32)

    def body(i, _):
        idx = idx_v[i]
        plsc.addupdate_scatter(deg_v, [zeros16i, idx], ones16)
        return 0

    lax.fori_loop(0, DEG_STEPS, body, 0)
    pltpu.sync_copy(deg_v, out_hbm.at[wid])


@functools.partial(
    pl.kernel,
    mesh=_mesh(),
    out_type=jax.ShapeDtypeStruct((NC, N_ACC, D), jnp.float32),
    scratch_types=[
        pltpu.VMEM((STEPS, GB), jnp.int32),
        pltpu.VMEM((STEPS, GB), jnp.int32),
        pltpu.VMEM((GB, D), jnp.float32),
        pltpu.VMEM_SHARED((N_ACC, D), jnp.float32),
        pltpu.SemaphoreType.DMA,
    ],
    compiler_params=pltpu.CompilerParams(needs_layout_passes=False),
)
def _scatter_sc(h_hbm, src_hbm, dst_hbm, zeros_hbm, out_hbm,
                src_v, dst_v, rows_v, acc_sh, sem):
    c = lax.axis_index("c")
    s = lax.axis_index("s")
    wid = c * NS + s
    pltpu.sync_copy(zeros_hbm, acc_sh.at[pl.ds(s * ROWS_PER_TILE, ROWS_PER_TILE)])
    pltpu.sync_copy(src_hbm.at[wid], src_v)
    pltpu.sync_copy(dst_hbm.at[wid], dst_v)
    plsc.subcore_barrier()

    def step(j, _):
        pltpu.async_copy(h_hbm.at[src_v.at[j]], rows_v, sem).wait()
        pltpu.sync_copy(rows_v, acc_sh.at[dst_v.at[j]], add=True)
        return 0

    lax.fori_loop(0, STEPS, step, 0)
    plsc.subcore_barrier()
    pltpu.sync_copy(
        acc_sh.at[pl.ds(s * ROWS_PER_TILE, ROWS_PER_TILE)],
        out_hbm.at[c, pl.ds(s * ROWS_PER_TILE, ROWS_PER_TILE)],
    )


RB = 1000
NBLK = N_NODES // RB


def _tc1_body(degp_ref, x_ref, w_ref, dinv_ref, h1p_ref):
    dinv = lax.rsqrt(jnp.sum(degp_ref[...], axis=1, keepdims=True) + 1.0)
    h = jnp.dot(x_ref[...], w_ref[...], preferred_element_type=jnp.float32)
    dinv_ref[...] = dinv
    h1p_ref[...] = h * dinv


def _tc1(deg_parts_t, x, W1):
    return pl.pallas_call(
        _tc1_body,
        grid=(NBLK,),
        in_specs=[
            pl.BlockSpec((RB, NW), lambda i: (i, 0)),
            pl.BlockSpec((RB, D), lambda i: (i, 0)),
            pl.BlockSpec((D, D), lambda i: (0, 0)),
        ],
        out_specs=[
            pl.BlockSpec((RB, 1), lambda i: (i, 0)),
            pl.BlockSpec((RB, D), lambda i: (i, 0)),
        ],
        out_shape=[
            jax.ShapeDtypeStruct((N_NODES, 1), jnp.float32),
            jax.ShapeDtypeStruct((N_NODES, D), jnp.float32),
        ],
    )(deg_parts_t, x, W1)


def _tc2_body(acc_ref, hp_ref, dinv_ref, b_ref, w_ref, out_ref):
    dinv = dinv_ref[...]
    z = dinv * (acc_ref[0] + acc_ref[1] + hp_ref[...]) + b_ref[...]
    z = jnp.maximum(z, 0.0)
    h = jnp.dot(z, w_ref[...], preferred_element_type=jnp.float32)
    out_ref[...] = h * dinv


def _tc2(acc, hp, dinv, b, W2):
    return pl.pallas_call(
        _tc2_body,
        grid=(NBLK,),
        in_specs=[
            pl.BlockSpec((NC, RB, D), lambda i: (0, i, 0)),
            pl.BlockSpec((RB, D), lambda i: (i, 0)),
            pl.BlockSpec((RB, 1), lambda i: (i, 0)),
            pl.BlockSpec((1, D), lambda i: (0, 0)),
            pl.BlockSpec((D, D), lambda i: (0, 0)),
        ],
        out_specs=pl.BlockSpec((RB, D), lambda i: (i, 0)),
        out_shape=jax.ShapeDtypeStruct((N_NODES, D), jnp.float32),
    )(acc, hp, dinv, b, W2)


def _tc3_body(acc_ref, hp_ref, dinv_ref, b_ref, wl_ref, bl_ref, out_ref):
    dinv = dinv_ref[...]
    z = dinv * (acc_ref[0] + acc_ref[1] + hp_ref[...]) + b_ref[...]
    z = jnp.maximum(z, 0.0)
    out = lax.dot_general(z, wl_ref[...], (((1,), (1,)), ((), ())),
                          preferred_element_type=jnp.float32)
    out_ref[...] = out + bl_ref[...]


def _tc3(acc, hp, dinv, b, Wl, bl):
    ncls = Wl.shape[0]
    return pl.pallas_call(
        _tc3_body,
        grid=(NBLK,),
        in_specs=[
            pl.BlockSpec((NC, RB, D), lambda i: (0, i, 0)),
            pl.BlockSpec((RB, D), lambda i: (i, 0)),
            pl.BlockSpec((RB, 1), lambda i: (i, 0)),
            pl.BlockSpec((1, D), lambda i: (0, 0)),
            pl.BlockSpec((ncls, D), lambda i: (0, 0)),
            pl.BlockSpec((1, ncls), lambda i: (0, 0)),
        ],
        out_specs=pl.BlockSpec((RB, ncls), lambda i: (i, 0)),
        out_shape=jax.ShapeDtypeStruct((N_NODES, ncls), jnp.float32),
    )(acc, hp, dinv, b, Wl, bl)


def kernel(x, edge_index, W1, b1, W2, b2, Wl, bl):
    src = edge_index[0].reshape(NW, STEPS, GB)
    dst = edge_index[1].reshape(NW, STEPS, GB)
    dst_deg = edge_index[1].reshape(NW, DEG_STEPS, 16)
    zeros = jnp.zeros((ROWS_PER_TILE, D), jnp.float32)

    deg_parts = _deg_sc(dst_deg)
    dinv, h1p = _tc1(deg_parts.reshape(NW, N_NODES).T, x, W1)
    acc1 = _scatter_sc(h1p, src, dst, zeros)
    h2p = _tc2(acc1, h1p, dinv, b1.reshape(1, D), W2)
    acc2 = _scatter_sc(h2p, src, dst, zeros)
    out = _tc3(acc2, h2p, dinv, b2.reshape(1, D), Wl, bl.reshape(1, Wl.shape[0]))
    return out


# speedup vs baseline: 29.1735x; 1.5097x over previous
import functools

import jax
import jax.numpy as jnp
from jax import lax
from jax.experimental import pallas as pl
from jax.experimental.pallas import tpu as pltpu
from jax.experimental.pallas import tpu_sc as plsc

N_NODES = 10000
N_EDGES = 320000
D = 128

NC = 2
NS = 16
NW = NC * NS
E_PER_TILE = N_EDGES // NW
GB = 80
STEPS = E_PER_TILE // GB
N_ACC = 10240
ROWS_PER_TILE = N_ACC // NS
DEG_STEPS = E_PER_TILE // 16

_mesh = lambda: plsc.VectorSubcoreMesh(core_axis_name="c", subcore_axis_name="s")


@functools.partial(
    pl.kernel,
    mesh=_mesh(),
    out_type=jax.ShapeDtypeStruct((NW, 1, N_NODES), jnp.float32),
    scratch_types=[
        pltpu.VMEM((DEG_STEPS, 16), jnp.int32),
        pltpu.VMEM((1, N_NODES), jnp.float32),
    ],
    compiler_params=pltpu.CompilerParams(needs_layout_passes=False),
)
def _deg_sc(dst_hbm, out_hbm, idx_v, deg_v):
    c = lax.axis_index("c")
    s = lax.axis_index("s")
    wid = c * NS + s
    pltpu.sync_copy(dst_hbm.at[wid], idx_v)

    zeros16 = jnp.zeros((16,), jnp.float32)

    def zbody(i, _):
        deg_v[0, pl.ds(i * 16, 16)] = zeros16
        return 0

    lax.fori_loop(0, N_NODES // 16, zbody, 0)

    ones16 = jnp.ones((16,), jnp.float32)
    zeros16i = jnp.zeros((16,), jnp.int32)

    def body(i, _):
        idx = idx_v[i]
        plsc.addupdate_scatter(deg_v, [zeros16i, idx], ones16)
        return 0

    lax.fori_loop(0, DEG_STEPS, body, 0)
    pltpu.sync_copy(deg_v, out_hbm.at[wid])


NSLOT = 4


@functools.partial(
    pl.kernel,
    mesh=_mesh(),
    out_type=jax.ShapeDtypeStruct((NC, NS, ROWS_PER_TILE, D), jnp.float32),
    scratch_types=[
        pltpu.VMEM((NSLOT, 2, GB), jnp.int32),
        pltpu.VMEM((NSLOT, GB, D), jnp.float32),
        pltpu.VMEM_SHARED((N_ACC, D), jnp.float32),
        pltpu.SemaphoreType.DMA((NSLOT,)),
        pltpu.SemaphoreType.DMA((NSLOT,)),
    ],
    compiler_params=pltpu.CompilerParams(needs_layout_passes=False),
)
def _scatter_sc(h_hbm, idx_hbm, zeros_hbm, out_hbm,
                idx_v, rows_v, acc_sh, isem, gsem):
    c = lax.axis_index("c")
    s = lax.axis_index("s")
    wid = c * NS + s
    pltpu.sync_copy(zeros_hbm, acc_sh.at[pl.ds(s * ROWS_PER_TILE, ROWS_PER_TILE)])

    def istart(j, slot):
        pltpu.async_copy(idx_hbm.at[wid, j], idx_v.at[slot], isem.at[slot])

    def iwait(j, slot):
        pltpu.make_async_copy(
            idx_hbm.at[wid, j], idx_v.at[slot], isem.at[slot]
        ).wait()

    def gstart(j, slot):
        pltpu.async_copy(h_hbm.at[idx_v.at[slot, 0]], rows_v.at[slot],
                         gsem.at[slot])

    def gwait(j, slot):
        pltpu.make_async_copy(
            h_hbm.at[idx_v.at[slot, 0]], rows_v.at[slot], gsem.at[slot]
        ).wait()

    def scat(j, slot):
        pltpu.sync_copy(rows_v.at[slot], acc_sh.at[idx_v.at[slot, 1]],
                        add=True)

    plsc.subcore_barrier()

    for k in range(NSLOT):
        istart(k, k)
    iwait(0, 0)
    gstart(0, 0)
    iwait(1, 1)
    gstart(1, 1)

    def step(j, _):
        slot = lax.rem(j, NSLOT)
        gwait(j, slot)
        scat(j, slot)

        @pl.when(j + NSLOT < STEPS)
        def _():
            istart(j + NSLOT, slot)

        @pl.when(j + 2 < STEPS)
        def _():
            slot2 = lax.rem(j + 2, NSLOT)
            iwait(j + 2, slot2)
            gstart(j + 2, slot2)

        return 0

    lax.fori_loop(0, STEPS, step, 0)
    plsc.subcore_barrier()
    pltpu.sync_copy(
        acc_sh.at[pl.ds(s * ROWS_PER_TILE, ROWS_PER_TILE)],
        out_hbm.at[c, s],
    )


RB = 1000
NBLK = N_NODES // RB


def _tc1_body(degp_ref, x_ref, w_ref, dinv_ref, h1p_ref):
    dinv = lax.rsqrt(jnp.sum(degp_ref[...], axis=1, keepdims=True) + 1.0)
    h = jnp.dot(x_ref[...], w_ref[...], preferred_element_type=jnp.float32)
    dinv_ref[...] = dinv
    h1p_ref[...] = h * dinv


def _tc1(deg_parts_t, x, W1):
    return pl.pallas_call(
        _tc1_body,
        grid=(NBLK,),
        in_specs=[
            pl.BlockSpec((RB, NW), lambda i: (i, 0)),
            pl.BlockSpec((RB, D), lambda i: (i, 0)),
            pl.BlockSpec((D, D), lambda i: (0, 0)),
        ],
        out_specs=[
            pl.BlockSpec((RB, 1), lambda i: (i, 0)),
            pl.BlockSpec((RB, D), lambda i: (i, 0)),
        ],
        out_shape=[
            jax.ShapeDtypeStruct((N_NODES, 1), jnp.float32),
            jax.ShapeDtypeStruct((N_NODES, D), jnp.float32),
        ],
    )(deg_parts_t, x, W1)


def _tc2_body(acc_ref, hp_ref, dinv_ref, b_ref, w_ref, out_ref):
    dinv = dinv_ref[...]
    z = dinv * (acc_ref[0] + acc_ref[1] + hp_ref[...]) + b_ref[...]
    z = jnp.maximum(z, 0.0)
    h = jnp.dot(z, w_ref[...], preferred_element_type=jnp.float32)
    out_ref[...] = h * dinv


def _tc2(acc, hp, dinv, b, W2):
    return pl.pallas_call(
        _tc2_body,
        grid=(NBLK,),
        in_specs=[
            pl.BlockSpec((NC, RB, D), lambda i: (0, i, 0)),
            pl.BlockSpec((RB, D), lambda i: (i, 0)),
            pl.BlockSpec((RB, 1), lambda i: (i, 0)),
            pl.BlockSpec((1, D), lambda i: (0, 0)),
            pl.BlockSpec((D, D), lambda i: (0, 0)),
        ],
        out_specs=pl.BlockSpec((RB, D), lambda i: (i, 0)),
        out_shape=jax.ShapeDtypeStruct((N_NODES, D), jnp.float32),
    )(acc, hp, dinv, b, W2)


def _tc3_body(acc_ref, hp_ref, dinv_ref, b_ref, wl_ref, bl_ref, out_ref):
    dinv = dinv_ref[...]
    z = dinv * (acc_ref[0] + acc_ref[1] + hp_ref[...]) + b_ref[...]
    z = jnp.maximum(z, 0.0)
    out = lax.dot_general(z, wl_ref[...], (((1,), (1,)), ((), ())),
                          preferred_element_type=jnp.float32)
    out_ref[...] = out + bl_ref[...]


def _tc3(acc, hp, dinv, b, Wl, bl):
    ncls = Wl.shape[0]
    return pl.pallas_call(
        _tc3_body,
        grid=(NBLK,),
        in_specs=[
            pl.BlockSpec((NC, RB, D), lambda i: (0, i, 0)),
            pl.BlockSpec((RB, D), lambda i: (i, 0)),
            pl.BlockSpec((RB, 1), lambda i: (i, 0)),
            pl.BlockSpec((1, D), lambda i: (0, 0)),
            pl.BlockSpec((ncls, D), lambda i: (0, 0)),
            pl.BlockSpec((1, ncls), lambda i: (0, 0)),
        ],
        out_specs=pl.BlockSpec((RB, ncls), lambda i: (i, 0)),
        out_shape=jax.ShapeDtypeStruct((N_NODES, ncls), jnp.float32),
    )(acc, hp, dinv, b, Wl, bl)


def kernel(x, edge_index, W1, b1, W2, b2, Wl, bl):
    src = edge_index[0].reshape(NW, STEPS, 1, GB)
    dst = edge_index[1].reshape(NW, STEPS, 1, GB)
    idx = jnp.concatenate([src, dst], axis=2)
    dst_deg = edge_index[1].reshape(NW, DEG_STEPS, 16)
    zeros = jnp.zeros((ROWS_PER_TILE, D), jnp.float32)

    deg_parts = _deg_sc(dst_deg)
    dinv, h1p = _tc1(deg_parts.reshape(NW, N_NODES).T, x, W1)
    acc1 = _scatter_sc(h1p, idx, zeros).reshape(NC, N_ACC, D)
    h2p = _tc2(acc1, h1p, dinv, b1.reshape(1, D), W2)
    acc2 = _scatter_sc(h2p, idx, zeros).reshape(NC, N_ACC, D)
    out = _tc3(acc2, h2p, dinv, b2.reshape(1, D), Wl, bl.reshape(1, Wl.shape[0]))
    return out


# speedup vs baseline: 29.9418x; 1.0263x over previous
import functools

import jax
import jax.numpy as jnp
from jax import lax
from jax.experimental import pallas as pl
from jax.experimental.pallas import tpu as pltpu
from jax.experimental.pallas import tpu_sc as plsc

N_NODES = 10000
N_EDGES = 320000
D = 128

NC = 2
NS = 16
NW = NC * NS
E_PER_TILE = N_EDGES // NW
GB = 80
STEPS = E_PER_TILE // GB
N_ACC = 10240
ROWS_PER_TILE = N_ACC // NS
DEG_STEPS = E_PER_TILE // 16

_mesh = lambda: plsc.VectorSubcoreMesh(core_axis_name="c", subcore_axis_name="s")


@functools.partial(
    pl.kernel,
    mesh=_mesh(),
    out_type=jax.ShapeDtypeStruct((NW, 1, N_NODES), jnp.float32),
    scratch_types=[
        pltpu.VMEM((DEG_STEPS, 16), jnp.int32),
        pltpu.VMEM((1, N_NODES), jnp.float32),
    ],
    compiler_params=pltpu.CompilerParams(needs_layout_passes=False),
)
def _deg_sc(dst_hbm, out_hbm, idx_v, deg_v):
    c = lax.axis_index("c")
    s = lax.axis_index("s")
    wid = c * NS + s
    pltpu.sync_copy(dst_hbm.at[wid], idx_v)

    zeros16 = jnp.zeros((16,), jnp.float32)

    def zbody(i, _):
        deg_v[0, pl.ds(i * 16, 16)] = zeros16
        return 0

    lax.fori_loop(0, N_NODES // 16, zbody, 0)

    ones16 = jnp.ones((16,), jnp.float32)
    zeros16i = jnp.zeros((16,), jnp.int32)

    def body(i, _):
        idx = idx_v[i]
        plsc.addupdate_scatter(deg_v, [zeros16i, idx], ones16)
        return 0

    lax.fori_loop(0, DEG_STEPS, body, 0)
    pltpu.sync_copy(deg_v, out_hbm.at[wid])


NSLOT = 4


@functools.partial(
    pl.kernel,
    mesh=_mesh(),
    out_type=jax.ShapeDtypeStruct((NC, NS, ROWS_PER_TILE, D), jnp.float32),
    scratch_types=[
        pltpu.VMEM((NSLOT, 2, 1, GB), jnp.int32),
        pltpu.VMEM((NSLOT, GB, D), jnp.float32),
        pltpu.VMEM_SHARED((N_ACC, D), jnp.float32),
        pltpu.SemaphoreType.DMA((NSLOT,)),
        pltpu.SemaphoreType.DMA((NSLOT,)),
    ],
    compiler_params=pltpu.CompilerParams(needs_layout_passes=False),
)
def _scatter_sc(h_hbm, idx_hbm, out_hbm, idx_v, rows_v, acc_sh, isem, gsem):
    c = lax.axis_index("c")
    s = lax.axis_index("s")
    wid = c * NS + s

    zeros16 = jnp.zeros((16,), jnp.float32)

    def zbody(i, _):
        rows_v[0, lax.div(i, 8), pl.ds(lax.rem(i, 8) * 16, 16)] = zeros16
        return 0

    lax.fori_loop(0, GB * 8, zbody, 0)
    for k in range(ROWS_PER_TILE // GB):
        pltpu.async_copy(
            rows_v.at[0], acc_sh.at[pl.ds(s * ROWS_PER_TILE + k * GB, GB)],
            isem.at[0])
    for k in range(ROWS_PER_TILE // GB):
        pltpu.make_async_copy(
            rows_v.at[0], acc_sh.at[pl.ds(s * ROWS_PER_TILE + k * GB, GB)],
            isem.at[0]).wait()

    def istart(j, slot):
        pltpu.async_copy(idx_hbm.at[:, wid, j], idx_v.at[slot], isem.at[slot])

    def iwait(j, slot):
        pltpu.make_async_copy(
            idx_hbm.at[:, wid, j], idx_v.at[slot], isem.at[slot]
        ).wait()

    def gstart(j, slot):
        pltpu.async_copy(h_hbm.at[idx_v.at[slot, 0, 0]], rows_v.at[slot],
                         gsem.at[slot])

    def gwait(j, slot):
        pltpu.make_async_copy(
            h_hbm.at[idx_v.at[slot, 0, 0]], rows_v.at[slot], gsem.at[slot]
        ).wait()

    def scat(j, slot):
        pltpu.sync_copy(rows_v.at[slot], acc_sh.at[idx_v.at[slot, 1, 0]],
                        add=True)

    plsc.subcore_barrier()

    for k in range(NSLOT):
        istart(k, k)
    iwait(0, 0)
    gstart(0, 0)
    iwait(1, 1)
    gstart(1, 1)

    def step(j, _):
        slot = lax.rem(j, NSLOT)
        gwait(j, slot)
        scat(j, slot)

        @pl.when(j + NSLOT < STEPS)
        def _():
            istart(j + NSLOT, slot)

        @pl.when(j + 2 < STEPS)
        def _():
            slot2 = lax.rem(j + 2, NSLOT)
            iwait(j + 2, slot2)
            gstart(j + 2, slot2)

        return 0

    lax.fori_loop(0, STEPS, step, 0)
    plsc.subcore_barrier()
    pltpu.sync_copy(
        acc_sh.at[pl.ds(s * ROWS_PER_TILE, ROWS_PER_TILE)],
        out_hbm.at[c, s],
    )


RB = 1000
NBLK = N_NODES // RB


def _tc1_body(degp_ref, x_ref, w_ref, dinv_ref, h1p_ref):
    dinv = lax.rsqrt(jnp.sum(degp_ref[...], axis=1, keepdims=True) + 1.0)
    h = jnp.dot(x_ref[...], w_ref[...], preferred_element_type=jnp.float32)
    dinv_ref[...] = dinv
    h1p_ref[...] = h * dinv


def _tc1(deg_parts_t, x, W1):
    return pl.pallas_call(
        _tc1_body,
        grid=(NBLK,),
        in_specs=[
            pl.BlockSpec((RB, NW), lambda i: (i, 0)),
            pl.BlockSpec((RB, D), lambda i: (i, 0)),
            pl.BlockSpec((D, D), lambda i: (0, 0)),
        ],
        out_specs=[
            pl.BlockSpec((RB, 1), lambda i: (i, 0)),
            pl.BlockSpec((RB, D), lambda i: (i, 0)),
        ],
        out_shape=[
            jax.ShapeDtypeStruct((N_NODES, 1), jnp.float32),
            jax.ShapeDtypeStruct((N_NODES, D), jnp.float32),
        ],
    )(deg_parts_t, x, W1)


def _tc2_body(acc_ref, hp_ref, dinv_ref, b_ref, w_ref, out_ref):
    dinv = dinv_ref[...]
    z = dinv * (acc_ref[0] + acc_ref[1] + hp_ref[...]) + b_ref[...]
    z = jnp.maximum(z, 0.0)
    h = jnp.dot(z, w_ref[...], preferred_element_type=jnp.float32)
    out_ref[...] = h * dinv


def _tc2(acc, hp, dinv, b, W2):
    return pl.pallas_call(
        _tc2_body,
        grid=(NBLK,),
        in_specs=[
            pl.BlockSpec((NC, RB, D), lambda i: (0, i, 0)),
            pl.BlockSpec((RB, D), lambda i: (i, 0)),
            pl.BlockSpec((RB, 1), lambda i: (i, 0)),
            pl.BlockSpec((1, D), lambda i: (0, 0)),
            pl.BlockSpec((D, D), lambda i: (0, 0)),
        ],
        out_specs=pl.BlockSpec((RB, D), lambda i: (i, 0)),
        out_shape=jax.ShapeDtypeStruct((N_NODES, D), jnp.float32),
    )(acc, hp, dinv, b, W2)


def _tc3_body(acc_ref, hp_ref, dinv_ref, b_ref, wl_ref, bl_ref, out_ref):
    dinv = dinv_ref[...]
    z = dinv * (acc_ref[0] + acc_ref[1] + hp_ref[...]) + b_ref[...]
    z = jnp.maximum(z, 0.0)
    out = lax.dot_general(z, wl_ref[...], (((1,), (1,)), ((), ())),
                          preferred_element_type=jnp.float32)
    out_ref[...] = out + bl_ref[...]


def _tc3(acc, hp, dinv, b, Wl, bl):
    ncls = Wl.shape[0]
    return pl.pallas_call(
        _tc3_body,
        grid=(NBLK,),
        in_specs=[
            pl.BlockSpec((NC, RB, D), lambda i: (0, i, 0)),
            pl.BlockSpec((RB, D), lambda i: (i, 0)),
            pl.BlockSpec((RB, 1), lambda i: (i, 0)),
            pl.BlockSpec((1, D), lambda i: (0, 0)),
            pl.BlockSpec((ncls, D), lambda i: (0, 0)),
            pl.BlockSpec((1, ncls), lambda i: (0, 0)),
        ],
        out_specs=pl.BlockSpec((RB, ncls), lambda i: (i, 0)),
        out_shape=jax.ShapeDtypeStruct((N_NODES, ncls), jnp.float32),
    )(acc, hp, dinv, b, Wl, bl)


def kernel(x, edge_index, W1, b1, W2, b2, Wl, bl):
    idx = edge_index.reshape(2, NW, STEPS, 1, GB)
    dst_deg = edge_index[1].reshape(NW, DEG_STEPS, 16)

    deg_parts = _deg_sc(dst_deg)
    dinv, h1p = _tc1(deg_parts.reshape(NW, N_NODES).T, x, W1)
    acc1 = _scatter_sc(h1p, idx).reshape(NC, N_ACC, D)
    h2p = _tc2(acc1, h1p, dinv, b1.reshape(1, D), W2)
    acc2 = _scatter_sc(h2p, idx).reshape(NC, N_ACC, D)
    out = _tc3(acc2, h2p, dinv, b2.reshape(1, D), Wl, bl.reshape(1, Wl.shape[0]))
    return out


# speedup vs baseline: 34.4244x; 1.1497x over previous
import functools

import jax
import jax.numpy as jnp
from jax import lax
from jax.experimental import pallas as pl
from jax.experimental.pallas import tpu as pltpu
from jax.experimental.pallas import tpu_sc as plsc

N_NODES = 10000
N_EDGES = 320000
D = 128

NC = 2
NS = 16
NW = NC * NS
E_PER_TILE = N_EDGES // NW
GB = 80
STEPS = E_PER_TILE // GB
N_ACC = 10240
ROWS_PER_TILE = N_ACC // NS
DEG_STEPS = E_PER_TILE // 16

_mesh = lambda: plsc.VectorSubcoreMesh(core_axis_name="c", subcore_axis_name="s")


@functools.partial(
    pl.kernel,
    mesh=_mesh(),
    out_type=jax.ShapeDtypeStruct((NW, 1, N_NODES), jnp.float32),
    scratch_types=[
        pltpu.VMEM((DEG_STEPS, 16), jnp.int32),
        pltpu.VMEM((1, N_NODES), jnp.float32),
    ],
    compiler_params=pltpu.CompilerParams(needs_layout_passes=False),
)
def _deg_sc(dst_hbm, out_hbm, idx_v, deg_v):
    c = lax.axis_index("c")
    s = lax.axis_index("s")
    wid = c * NS + s
    pltpu.sync_copy(dst_hbm.at[wid], idx_v)

    zeros16 = jnp.zeros((16,), jnp.float32)

    def zbody(i, _):
        deg_v[0, pl.ds(i * 16, 16)] = zeros16
        return 0

    lax.fori_loop(0, N_NODES // 16, zbody, 0)

    ones16 = jnp.ones((16,), jnp.float32)
    zeros16i = jnp.zeros((16,), jnp.int32)

    def body(i, _):
        idx = idx_v[i]
        plsc.addupdate_scatter(deg_v, [zeros16i, idx], ones16)
        return 0

    lax.fori_loop(0, DEG_STEPS, body, 0)
    pltpu.sync_copy(deg_v, out_hbm.at[wid])


NSLOT = 4


@functools.partial(
    pl.kernel,
    mesh=_mesh(),
    out_type=jax.ShapeDtypeStruct((NC, NS, ROWS_PER_TILE, D), jnp.float32),
    scratch_types=[
        pltpu.VMEM((NSLOT, 2, 1, GB), jnp.int32),
        pltpu.VMEM((NSLOT, GB, D), jnp.float32),
        pltpu.VMEM_SHARED((N_ACC, D), jnp.float32),
        pltpu.SemaphoreType.DMA((NSLOT,)),
        pltpu.SemaphoreType.DMA((NSLOT,)),
    ],
    compiler_params=pltpu.CompilerParams(needs_layout_passes=False),
)
def _scatter_sc(h_hbm, idx_hbm, out_hbm, idx_v, rows_v, acc_sh, isem, gsem):
    c = lax.axis_index("c")
    s = lax.axis_index("s")
    wid = c * NS + s

    zeros16 = jnp.zeros((16,), jnp.float32)

    def zbody(i, _):
        rows_v[0, lax.div(i, 8), pl.ds(lax.rem(i, 8) * 16, 16)] = zeros16
        return 0

    lax.fori_loop(0, GB * 8, zbody, 0)
    for k in range(ROWS_PER_TILE // GB):
        pltpu.async_copy(
            rows_v.at[0], acc_sh.at[pl.ds(s * ROWS_PER_TILE + k * GB, GB)],
            isem.at[0])
    for k in range(ROWS_PER_TILE // GB):
        pltpu.make_async_copy(
            rows_v.at[0], acc_sh.at[pl.ds(s * ROWS_PER_TILE + k * GB, GB)],
            isem.at[0]).wait()

    def istart(j, slot):
        pltpu.async_copy(idx_hbm.at[:, wid, j], idx_v.at[slot], isem.at[slot])

    def iwait(j, slot):
        pltpu.make_async_copy(
            idx_hbm.at[:, wid, j], idx_v.at[slot], isem.at[slot]
        ).wait()

    def gstart(j, slot):
        pltpu.async_copy(h_hbm.at[idx_v.at[slot, 0, 0]], rows_v.at[slot],
                         gsem.at[slot])

    def gwait(j, slot):
        pltpu.make_async_copy(
            h_hbm.at[idx_v.at[slot, 0, 0]], rows_v.at[slot], gsem.at[slot]
        ).wait()

    def scat(j, slot):
        pltpu.sync_copy(rows_v.at[slot], acc_sh.at[idx_v.at[slot, 1, 0]],
                        add=True)

    plsc.subcore_barrier()

    for k in range(NSLOT):
        istart(k, k)
    for k in range(3):
        iwait(k, k)
        gstart(k, k)

    def step(j, _):
        slot = lax.rem(j, NSLOT)
        gwait(j, slot)
        scat(j, slot)

        @pl.when(j + NSLOT < STEPS)
        def _():
            istart(j + NSLOT, slot)

        @pl.when(j + 3 < STEPS)
        def _():
            slot3 = lax.rem(j + 3, NSLOT)
            iwait(j + 3, slot3)
            gstart(j + 3, slot3)

        return 0

    lax.fori_loop(0, STEPS, step, 0)
    plsc.subcore_barrier()
    pltpu.sync_copy(
        acc_sh.at[pl.ds(s * ROWS_PER_TILE, ROWS_PER_TILE)],
        out_hbm.at[c, s],
    )


RB = 1000
NBLK = N_NODES // RB


def _tc1_body(degp_ref, x_ref, w_ref, dinv_ref, h1p_ref):
    dinv = lax.rsqrt(jnp.sum(degp_ref[...], axis=1, keepdims=True) + 1.0)
    h = jnp.dot(x_ref[...], w_ref[...], preferred_element_type=jnp.float32)
    dinv_ref[...] = dinv
    h1p_ref[...] = h * dinv


def _tc1(deg_parts_t, x, W1):
    return pl.pallas_call(
        _tc1_body,
        grid=(NBLK,),
        in_specs=[
            pl.BlockSpec((RB, NW), lambda i: (i, 0)),
            pl.BlockSpec((RB, D), lambda i: (i, 0)),
            pl.BlockSpec((D, D), lambda i: (0, 0)),
        ],
        out_specs=[
            pl.BlockSpec((RB, 1), lambda i: (i, 0)),
            pl.BlockSpec((RB, D), lambda i: (i, 0)),
        ],
        out_shape=[
            jax.ShapeDtypeStruct((N_NODES, 1), jnp.float32),
            jax.ShapeDtypeStruct((N_NODES, D), jnp.float32),
        ],
    )(deg_parts_t, x, W1)


def _tc2_body(acc_ref, hp_ref, dinv_ref, b_ref, w_ref, out_ref):
    dinv = dinv_ref[...]
    z = dinv * (acc_ref[0] + acc_ref[1] + hp_ref[...]) + b_ref[...]
    z = jnp.maximum(z, 0.0)
    h = jnp.dot(z, w_ref[...], preferred_element_type=jnp.float32)
    out_ref[...] = h * dinv


def _tc2(acc, hp, dinv, b, W2):
    return pl.pallas_call(
        _tc2_body,
        grid=(NBLK,),
        in_specs=[
            pl.BlockSpec((NC, RB, D), lambda i: (0, i, 0)),
            pl.BlockSpec((RB, D), lambda i: (i, 0)),
            pl.BlockSpec((RB, 1), lambda i: (i, 0)),
            pl.BlockSpec((1, D), lambda i: (0, 0)),
            pl.BlockSpec((D, D), lambda i: (0, 0)),
        ],
        out_specs=pl.BlockSpec((RB, D), lambda i: (i, 0)),
        out_shape=jax.ShapeDtypeStruct((N_NODES, D), jnp.float32),
    )(acc, hp, dinv, b, W2)


def _tc3_body(acc_ref, hp_ref, dinv_ref, b_ref, wl_ref, bl_ref, out_ref):
    dinv = dinv_ref[...]
    z = dinv * (acc_ref[0] + acc_ref[1] + hp_ref[...]) + b_ref[...]
    z = jnp.maximum(z, 0.0)
    out = lax.dot_general(z, wl_ref[...], (((1,), (1,)), ((), ())),
                          preferred_element_type=jnp.float32)
    out_ref[...] = out + bl_ref[...]


def _tc3(acc, hp, dinv, b, Wl, bl):
    ncls = Wl.shape[0]
    return pl.pallas_call(
        _tc3_body,
        grid=(NBLK,),
        in_specs=[
            pl.BlockSpec((NC, RB, D), lambda i: (0, i, 0)),
            pl.BlockSpec((RB, D), lambda i: (i, 0)),
            pl.BlockSpec((RB, 1), lambda i: (i, 0)),
            pl.BlockSpec((1, D), lambda i: (0, 0)),
            pl.BlockSpec((ncls, D), lambda i: (0, 0)),
            pl.BlockSpec((1, ncls), lambda i: (0, 0)),
        ],
        out_specs=pl.BlockSpec((RB, ncls), lambda i: (i, 0)),
        out_shape=jax.ShapeDtypeStruct((N_NODES, ncls), jnp.float32),
    )(acc, hp, dinv, b, Wl, bl)


def kernel(x, edge_index, W1, b1, W2, b2, Wl, bl):
    idx = edge_index.reshape(2, NW, STEPS, 1, GB)
    dst_deg = edge_index[1].reshape(NW, DEG_STEPS, 16)

    deg_parts = _deg_sc(dst_deg)
    dinv, h1p = _tc1(deg_parts.reshape(NW, N_NODES).T, x, W1)
    acc1 = _scatter_sc(h1p, idx).reshape(NC, N_ACC, D)
    h2p = _tc2(acc1, h1p, dinv, b1.reshape(1, D), W2)
    acc2 = _scatter_sc(h2p, idx).reshape(NC, N_ACC, D)
    out = _tc3(acc2, h2p, dinv, b2.reshape(1, D), Wl, bl.reshape(1, Wl.shape[0]))
    return out


# speedup vs baseline: 34.8235x; 1.0116x over previous
import functools

import jax
import jax.numpy as jnp
from jax import lax
from jax.experimental import pallas as pl
from jax.experimental.pallas import tpu as pltpu
from jax.experimental.pallas import tpu_sc as plsc

N_NODES = 10000
N_EDGES = 320000
D = 128

NC = 2
NS = 16
NW = NC * NS
E_PER_TILE = N_EDGES // NW
GB = 80
STEPS = E_PER_TILE // GB
N_ACC = 10240
ROWS_PER_TILE = N_ACC // NS
DEG_STEPS = E_PER_TILE // 16

_mesh = lambda: plsc.VectorSubcoreMesh(core_axis_name="c", subcore_axis_name="s")


DEG_R = N_ACC // 16


@functools.partial(
    pl.kernel,
    mesh=_mesh(),
    out_type=jax.ShapeDtypeStruct((NW, DEG_R, 16), jnp.float32),
    scratch_types=[
        pltpu.VMEM((STEPS, 1, GB), jnp.int32),
        pltpu.VMEM((DEG_R, 16), jnp.float32),
        pltpu.VMEM((DEG_R // 128, 128), jnp.int32),
        pltpu.VMEM_SHARED((DEG_R, 16), jnp.float32),
    ],
    compiler_params=pltpu.CompilerParams(needs_layout_passes=False),
)
def _deg_sc(idx_hbm, out_hbm, dst_v, deg_v, ramp_v, spdeg):
    c = lax.axis_index("c")
    s = lax.axis_index("s")
    wid = c * NS + s
    pltpu.sync_copy(idx_hbm.at[1, wid], dst_v)

    zeros16 = jnp.zeros((16,), jnp.float32)
    iota16 = lax.iota(jnp.int32, 16)

    def zbody(i, _):
        deg_v[i, :] = zeros16
        return 0

    lax.fori_loop(0, DEG_R, zbody, 0)
    for k in range(DEG_R // 128):
        for i in range(8):
            ramp_v[k, pl.ds(i * 16, 16)] = iota16 + (k * 128 + i * 16)
    rpt = DEG_R // NS
    pltpu.sync_copy(deg_v.at[pl.ds(s * rpt, rpt)], spdeg.at[pl.ds(s * rpt, rpt)])
    plsc.subcore_barrier()

    ones16 = jnp.ones((16,), jnp.float32)

    def body(r, _):
        for k in range(GB // 16):
            d = dst_v[r, 0, pl.ds(k * 16, 16)]
            row = lax.shift_right_logical(d, 4)
            col = lax.bitwise_and(d, 15)
            plsc.addupdate_scatter(deg_v, [row, col], ones16)
        return 0

    lax.fori_loop(0, STEPS, body, 0)
    pltpu.sync_copy(deg_v, out_hbm.at[wid])


NSLOT = 4


@functools.partial(
    pl.kernel,
    mesh=_mesh(),
    out_type=jax.ShapeDtypeStruct((NC, NS, ROWS_PER_TILE, D), jnp.float32),
    scratch_types=[
        pltpu.VMEM((NSLOT, 2, 1, GB), jnp.int32),
        pltpu.VMEM((NSLOT, GB, D), jnp.float32),
        pltpu.VMEM_SHARED((N_ACC, D), jnp.float32),
        pltpu.SemaphoreType.DMA((NSLOT,)),
        pltpu.SemaphoreType.DMA((NSLOT,)),
    ],
    compiler_params=pltpu.CompilerParams(needs_layout_passes=False),
)
def _scatter_sc(h_hbm, idx_hbm, out_hbm, idx_v, rows_v, acc_sh, isem, gsem):
    c = lax.axis_index("c")
    s = lax.axis_index("s")
    wid = c * NS + s

    zeros16 = jnp.zeros((16,), jnp.float32)

    def zbody(i, _):
        rows_v[0, lax.div(i, 8), pl.ds(lax.rem(i, 8) * 16, 16)] = zeros16
        return 0

    lax.fori_loop(0, GB * 8, zbody, 0)
    for k in range(ROWS_PER_TILE // GB):
        pltpu.async_copy(
            rows_v.at[0], acc_sh.at[pl.ds(s * ROWS_PER_TILE + k * GB, GB)],
            isem.at[0])
    for k in range(ROWS_PER_TILE // GB):
        pltpu.make_async_copy(
            rows_v.at[0], acc_sh.at[pl.ds(s * ROWS_PER_TILE + k * GB, GB)],
            isem.at[0]).wait()

    def istart(j, slot):
        pltpu.async_copy(idx_hbm.at[:, wid, j], idx_v.at[slot], isem.at[slot])

    def iwait(j, slot):
        pltpu.make_async_copy(
            idx_hbm.at[:, wid, j], idx_v.at[slot], isem.at[slot]
        ).wait()

    def gstart(j, slot):
        pltpu.async_copy(h_hbm.at[idx_v.at[slot, 0, 0]], rows_v.at[slot],
                         gsem.at[slot])

    def gwait(j, slot):
        pltpu.make_async_copy(
            h_hbm.at[idx_v.at[slot, 0, 0]], rows_v.at[slot], gsem.at[slot]
        ).wait()

    def scat(j, slot):
        pltpu.sync_copy(rows_v.at[slot], acc_sh.at[idx_v.at[slot, 1, 0]],
                        add=True)

    plsc.subcore_barrier()

    for k in range(NSLOT):
        istart(k, k)
    for k in range(3):
        iwait(k, k)
        gstart(k, k)

    def step(j, _):
        slot = lax.rem(j, NSLOT)
        gwait(j, slot)
        scat(j, slot)

        @pl.when(j + NSLOT < STEPS)
        def _():
            istart(j + NSLOT, slot)

        @pl.when(j + 3 < STEPS)
        def _():
            slot3 = lax.rem(j + 3, NSLOT)
            iwait(j + 3, slot3)
            gstart(j + 3, slot3)

        return 0

    lax.fori_loop(0, STEPS, step, 0)
    plsc.subcore_barrier()
    pltpu.sync_copy(
        acc_sh.at[pl.ds(s * ROWS_PER_TILE, ROWS_PER_TILE)],
        out_hbm.at[c, s],
    )


RB = 1000
NBLK = N_NODES // RB


def _tc1_body(degp_ref, x_ref, w_ref, dinv_ref, h1p_ref):
    dinv = lax.rsqrt(degp_ref[...] + 1.0)
    h = jnp.dot(x_ref[...], w_ref[...], preferred_element_type=jnp.float32)
    dinv_ref[...] = dinv
    h1p_ref[...] = h * dinv


def _tc1(deg_col, x, W1):
    return pl.pallas_call(
        _tc1_body,
        grid=(NBLK,),
        in_specs=[
            pl.BlockSpec((RB, 1), lambda i: (i, 0)),
            pl.BlockSpec((RB, D), lambda i: (i, 0)),
            pl.BlockSpec((D, D), lambda i: (0, 0)),
        ],
        out_specs=[
            pl.BlockSpec((RB, 1), lambda i: (i, 0)),
            pl.BlockSpec((RB, D), lambda i: (i, 0)),
        ],
        out_shape=[
            jax.ShapeDtypeStruct((N_NODES, 1), jnp.float32),
            jax.ShapeDtypeStruct((N_NODES, D), jnp.float32),
        ],
    )(deg_col, x, W1)


def _tc2_body(acc_ref, hp_ref, dinv_ref, b_ref, w_ref, out_ref):
    dinv = dinv_ref[...]
    z = dinv * (acc_ref[0] + acc_ref[1] + hp_ref[...]) + b_ref[...]
    z = jnp.maximum(z, 0.0)
    h = jnp.dot(z, w_ref[...], preferred_element_type=jnp.float32)
    out_ref[...] = h * dinv


def _tc2(acc, hp, dinv, b, W2):
    return pl.pallas_call(
        _tc2_body,
        grid=(NBLK,),
        in_specs=[
            pl.BlockSpec((NC, RB, D), lambda i: (0, i, 0)),
            pl.BlockSpec((RB, D), lambda i: (i, 0)),
            pl.BlockSpec((RB, 1), lambda i: (i, 0)),
            pl.BlockSpec((1, D), lambda i: (0, 0)),
            pl.BlockSpec((D, D), lambda i: (0, 0)),
        ],
        out_specs=pl.BlockSpec((RB, D), lambda i: (i, 0)),
        out_shape=jax.ShapeDtypeStruct((N_NODES, D), jnp.float32),
    )(acc, hp, dinv, b, W2)


def _tc3_body(acc_ref, hp_ref, dinv_ref, b_ref, wl_ref, bl_ref, out_ref):
    dinv = dinv_ref[...]
    z = dinv * (acc_ref[0] + acc_ref[1] + hp_ref[...]) + b_ref[...]
    z = jnp.maximum(z, 0.0)
    out = lax.dot_general(z, wl_ref[...], (((1,), (1,)), ((), ())),
                          preferred_element_type=jnp.float32)
    out_ref[...] = out + bl_ref[...]


def _tc3(acc, hp, dinv, b, Wl, bl):
    ncls = Wl.shape[0]
    return pl.pallas_call(
        _tc3_body,
        grid=(NBLK,),
        in_specs=[
            pl.BlockSpec((NC, RB, D), lambda i: (0, i, 0)),
            pl.BlockSpec((RB, D), lambda i: (i, 0)),
            pl.BlockSpec((RB, 1), lambda i: (i, 0)),
            pl.BlockSpec((1, D), lambda i: (0, 0)),
            pl.BlockSpec((ncls, D), lambda i: (0, 0)),
            pl.BlockSpec((1, ncls), lambda i: (0, 0)),
        ],
        out_specs=pl.BlockSpec((RB, ncls), lambda i: (i, 0)),
        out_shape=jax.ShapeDtypeStruct((N_NODES, ncls), jnp.float32),
    )(acc, hp, dinv, b, Wl, bl)


def kernel(x, edge_index, W1, b1, W2, b2, Wl, bl):
    idx = edge_index.reshape(2, NW, STEPS, 1, GB)

    deg_parts = _deg_sc(idx)
    deg_col = deg_parts.sum(axis=0).reshape(N_ACC)[:N_NODES]
    dinv, h1p = _tc1(deg_col.reshape(N_NODES, 1), x, W1)
    acc1 = _scatter_sc(h1p, idx).reshape(NC, N_ACC, D)
    h2p = _tc2(acc1, h1p, dinv, b1.reshape(1, D), W2)
    acc2 = _scatter_sc(h2p, idx).reshape(NC, N_ACC, D)
    out = _tc3(acc2, h2p, dinv, b2.reshape(1, D), Wl, bl.reshape(1, Wl.shape[0]))
    return out
